# baseline (speedup 1.0000x reference)
import functools

import jax
import jax.numpy as jnp
from jax import lax
from jax.experimental import pallas as pl
from jax.experimental.pallas import tpu as pltpu
from jax.experimental.pallas import tpu_sc as plsc

_B = 16384
_D = 16
_NC = 2
_NS = 16
_NW = _NC * _NS
_BPW = _B // _NW
_CH = 128
_NCH = _BPW // _CH


def _gather_body(u_tab, i_tab, uidx, iidx, u_out, i_out,
                 uidx_v, iidx_v, u_rows, i_rows, sem):
    wid = lax.axis_index("s") * _NC + lax.axis_index("c")
    base = wid * _BPW
    cbase = wid * _NCH
    pltpu.sync_copy(uidx.at[pl.ds(cbase, _NCH)], uidx_v)
    pltpu.sync_copy(iidx.at[pl.ds(cbase, _NCH)], iidx_v)
    copies = []
    for j in range(_NCH):
        copies.append(pltpu.async_copy(
            u_tab.at[uidx_v.at[j]], u_rows.at[pl.ds(j * _CH, _CH)], sem))
        copies.append(pltpu.async_copy(
            i_tab.at[iidx_v.at[j]], i_rows.at[pl.ds(j * _CH, _CH)], sem))
    for c in copies:
        c.wait()
    pltpu.sync_copy(u_rows, u_out.at[pl.ds(base, _BPW)])
    pltpu.sync_copy(i_rows, i_out.at[pl.ds(base, _BPW)])


_gather2 = functools.partial(
    pl.kernel,
    mesh=plsc.VectorSubcoreMesh(core_axis_name="c", subcore_axis_name="s"),
    out_type=(jax.ShapeDtypeStruct((_B, _D), jnp.float32),
              jax.ShapeDtypeStruct((_B, _D), jnp.float32)),
    scratch_types=[
        pltpu.VMEM((_NCH, _CH), jnp.int32),
        pltpu.VMEM((_NCH, _CH), jnp.int32),
        pltpu.VMEM((_BPW, _D), jnp.float32),
        pltpu.VMEM((_BPW, _D), jnp.float32),
        pltpu.SemaphoreType.DMA,
    ],
    compiler_params=pltpu.CompilerParams(use_tc_tiling_on_sc=False),
)(_gather_body)


def _mlp_body(u_ref, i_ref, w1u_ref, w1i_ref, b1_ref, w2_ref, out_ref):
    h = u_ref[...] @ w1u_ref[...] + i_ref[...] @ w1i_ref[...] + b1_ref[...]
    out_ref[...] = jnp.maximum(h, 0.0) @ w2_ref[...]


_mlp = pl.pallas_call(
    _mlp_body,
    out_shape=jax.ShapeDtypeStruct((_B, 1), jnp.float32),
)


def kernel(x, user_table, item_table, W1, b1, W2):
    uidx = x[:, 0].reshape(_B // _CH, _CH)
    iidx = x[:, 1].reshape(_B // _CH, _CH)
    u_emb, i_emb = _gather2(user_table, item_table, uidx, iidx)
    out = _mlp(u_emb, i_emb, W1[:_D], W1[_D:], b1.reshape(1, _D), W2)
    return (out, u_emb, i_emb)
